# Initial kernel scaffold; baseline (speedup 1.0000x reference)
import jax
import jax.numpy as jnp
from jax.experimental import pallas as pl


def kernel(cost_maps, start_maps, goal_maps, heuristic_maps, obstacles_maps):
    raise NotImplementedError("write your pallas kernel here")


# speedup vs baseline: 765.2887x; 765.2887x over previous
import functools
import math

import jax
import jax.numpy as jnp
from jax import lax
from jax.experimental import pallas as pl
from jax.experimental.pallas import tpu as pltpu
from jax.experimental.pallas import tpu_sc as plsc

B, H, W = 8, 32, 32
HW = H * W
TMAX = HW
SQW = math.sqrt(W)
NC, NS = 2, 16
L = 16


def _astar_body(cm_hbm, h_hbm, gm_hbm, sm_hbm, hist_hbm, path_hbm,
                cm_v, h_v, gm_v, sm_v, val_v, g_v, open_v, hist_v,
                parents_v, path_v):
    wid = lax.axis_index("s") * NC + lax.axis_index("c")

    @pl.when(wid < B)
    def _():
        b = wid
        pltpu.sync_copy(cm_hbm.at[b], cm_v)
        pltpu.sync_copy(h_hbm.at[b], h_v)
        pltpu.sync_copy(gm_hbm.at[b], gm_v)
        pltpu.sync_copy(sm_hbm.at[b], sm_v)

        lane = lax.iota(jnp.int32, L)
        zero_f = jnp.zeros((L,), jnp.float32)
        one_f = jnp.ones((L,), jnp.float32)
        one_i = jnp.ones((L,), jnp.int32)
        lane0 = lane == 0

        def goal_chunk(c, acc):
            s = c * L
            gmc = gm_v[pl.ds(s, L)]
            cand = jnp.where(gmc > 0.5, -(s + lane).astype(jnp.float32),
                             -float(HW))
            return jnp.maximum(acc, jnp.max(cand))

        goal_i = (-lax.fori_loop(0, HW // L, goal_chunk,
                                 jnp.float32(-HW))).astype(jnp.int32)

        def init_chunk(c, _):
            s = c * L
            hc = h_v[pl.ds(s, L)]
            smc = sm_v[pl.ds(s, L)]
            gmc = gm_v[pl.ds(s, L)]
            g_v[pl.ds(s, L)] = zero_f
            hist_v[pl.ds(s, L)] = zero_f
            open_v[pl.ds(s, L)] = smc
            f0 = 0.5 * hc
            val_v[pl.ds(s, L)] = jnp.exp(-f0 / SQW) * smc
            parents_v[pl.ds(s, L)] = jnp.full((L,), goal_i, jnp.int32)
            path_v[pl.ds(s, L)] = (gmc > 0.5).astype(jnp.int32)
            return 0

        lax.fori_loop(0, HW // L, init_chunk, 0)

        nk = lane + (lane >= 4).astype(jnp.int32)
        dy = nk // 3 - 1
        dx = nk % 3 - 1
        nbr_lane = lane < 8

        def cond(carry):
            t, solved, _ = carry
            return jnp.logical_and(t < TMAX, jnp.logical_not(solved))

        def body(carry):
            t, _, t1 = carry

            def amax_chunk(c, st):
                bv, bi = st
                s = c * L
                v = val_v[pl.ds(s, L)]
                m = v > bv
                return jnp.where(m, v, bv), jnp.where(m, s + lane, bi)

            bv, bi = lax.fori_loop(
                0, HW // L, amax_chunk,
                (jnp.full((L,), -1.0, jnp.float32), jnp.zeros((L,), jnp.int32)))
            maxv = jnp.max(bv)
            ind = (-jnp.max(jnp.where(bv == maxv, -bi.astype(jnp.float32),
                                      -float(HW)))).astype(jnp.int32)
            indv = jnp.full((L,), ind, jnp.int32)
            solved = ind == goal_i

            plsc.store_scatter(hist_v, [indv], one_f, mask=lane0)
            rm = jnp.logical_and(lane0, jnp.logical_not(solved))
            plsc.store_scatter(open_v, [indv], zero_f, mask=rm)
            plsc.store_scatter(val_v, [indv], zero_f, mask=rm)

            gsel = plsc.load_gather(g_v, [indv])
            csel = plsc.load_gather(cm_v, [indv])
            new_g = jnp.max(gsel) + jnp.max(csel)
            new_gv = jnp.full((L,), new_g, jnp.float32)

            ny = (indv >> 5) + dy
            nx = (indv & 31) + dx
            valid = (nbr_lane & (ny >= 0) & (ny < H) & (nx >= 0) & (nx < W))
            nidx = jnp.where(valid, ny * W + nx, 0)
            gn = plsc.load_gather(g_v, [nidx], mask=valid)
            on = plsc.load_gather(open_v, [nidx], mask=valid)
            hn = plsc.load_gather(hist_v, [nidx], mask=valid)
            hh = plsc.load_gather(h_v, [nidx], mask=valid)
            upd = valid & (((on <= 0.5) & (hn <= 0.5))
                           | ((on > 0.5) & (gn > new_gv)))
            plsc.store_scatter(g_v, [nidx], new_gv, mask=upd)
            plsc.store_scatter(open_v, [nidx], one_f, mask=upd)
            plsc.store_scatter(parents_v, [nidx], indv, mask=upd)
            fn = 0.5 * new_gv + 0.5 * hh
            plsc.store_scatter(val_v, [nidx], jnp.exp(-fn / SQW), mask=upd)

            t1 = jnp.where(solved, t, t1)
            return t + 1, solved, t1

        _, _, t1 = lax.while_loop(
            cond, body,
            (jnp.int32(0), jnp.bool_(False), jnp.int32(TMAX - 1)))

        goalv = jnp.full((L,), goal_i, jnp.int32)
        loc0 = jnp.max(plsc.load_gather(parents_v,
                                        [goalv]).astype(jnp.float32))

        def bt(_, loc):
            locv = jnp.full((L,), loc.astype(jnp.int32), jnp.int32)
            plsc.store_scatter(path_v, [locv], one_i, mask=lane0)
            return jnp.max(plsc.load_gather(parents_v,
                                            [locv]).astype(jnp.float32))

        lax.fori_loop(0, t1, bt, loc0)

        pltpu.sync_copy(hist_v, hist_hbm.at[b])
        pltpu.sync_copy(path_v, path_hbm.at[b])


@jax.jit
def _astar_sc(cm, h, gm, sm):
    mesh = plsc.VectorSubcoreMesh(core_axis_name="c", subcore_axis_name="s",
                                  num_cores=NC, num_subcores=NS)
    f32 = jnp.float32
    run = pl.kernel(
        _astar_body,
        out_type=(jax.ShapeDtypeStruct((B, HW), f32),
                  jax.ShapeDtypeStruct((B, HW), jnp.int32)),
        mesh=mesh,
        compiler_params=pltpu.CompilerParams(needs_layout_passes=False),
        scratch_types=(
            pltpu.VMEM((HW,), f32),
            pltpu.VMEM((HW,), f32),
            pltpu.VMEM((HW,), f32),
            pltpu.VMEM((HW,), f32),
            pltpu.VMEM((HW,), f32),
            pltpu.VMEM((HW,), f32),
            pltpu.VMEM((HW,), f32),
            pltpu.VMEM((HW,), f32),
            pltpu.VMEM((HW,), jnp.int32),
            pltpu.VMEM((HW,), jnp.int32),
        ),
    )
    return run(cm, h, gm, sm)


def _heuristic(gm):
    gy, gx = jnp.meshgrid(jnp.arange(H), jnp.arange(W), indexing="ij")
    loc = jnp.stack([gy, gx], 0).astype(gm.dtype)
    loc_e = loc.reshape(2, -1)[None]
    goal_loc = jnp.einsum("kij,bij->bk", loc, gm)[:, :, None]
    dxdy = jnp.abs(loc_e - goal_loc)
    h = dxdy.sum(1) - dxdy.min(1)
    euc = jnp.sqrt(((loc_e - goal_loc) ** 2).sum(1))
    return h + 0.001 * euc


def kernel(cost_maps, start_maps, goal_maps, heuristic_maps, obstacles_maps):
    cm = cost_maps[:, 0].reshape(B, HW)
    gm2 = goal_maps[:, 0]
    gm = gm2.reshape(B, HW)
    sm = start_maps[:, 0].reshape(B, HW)
    h = _heuristic(gm2)
    hist, path = _astar_sc(cm, h, gm, sm)
    return hist.reshape(B, 1, H, W), path.reshape(B, 1, H, W)


# speedup vs baseline: 803.2136x; 1.0496x over previous
import functools
import math

import jax
import jax.numpy as jnp
from jax import lax
from jax.experimental import pallas as pl
from jax.experimental.pallas import tpu as pltpu
from jax.experimental.pallas import tpu_sc as plsc

B, H, W = 8, 32, 32
HW = H * W
TMAX = HW
SQW = math.sqrt(W)
NC, NS = 1, 16
L = 16


def _astar_body(cm_hbm, h_hbm, gm_hbm, sm_hbm, hist_hbm, path_hbm,
                cm_v, h_v, gm_v, sm_v, val_v, g_v, open_v, hist_v,
                parents_v, path_v):
    wid = lax.axis_index("s") * NC + lax.axis_index("c")

    @pl.when(wid < B)
    def _():
        b = wid
        pltpu.sync_copy(cm_hbm.at[b], cm_v)
        pltpu.sync_copy(h_hbm.at[b], h_v)
        pltpu.sync_copy(gm_hbm.at[b], gm_v)
        pltpu.sync_copy(sm_hbm.at[b], sm_v)

        lane = lax.iota(jnp.int32, L)
        zero_f = jnp.zeros((L,), jnp.float32)
        one_f = jnp.ones((L,), jnp.float32)
        one_i = jnp.ones((L,), jnp.int32)
        lane0 = lane == 0

        def goal_chunk(c, acc):
            s = c * L
            gmc = gm_v[pl.ds(s, L)]
            cand = jnp.where(gmc > 0.5, -(s + lane).astype(jnp.float32),
                             -float(HW))
            return jnp.maximum(acc, jnp.max(cand))

        goal_i = (-lax.fori_loop(0, HW // L, goal_chunk,
                                 jnp.float32(-HW))).astype(jnp.int32)

        def init_chunk(c, _):
            s = c * L
            hc = h_v[pl.ds(s, L)]
            smc = sm_v[pl.ds(s, L)]
            gmc = gm_v[pl.ds(s, L)]
            g_v[pl.ds(s, L)] = zero_f
            hist_v[pl.ds(s, L)] = zero_f
            open_v[pl.ds(s, L)] = smc
            f0 = 0.5 * hc
            val_v[pl.ds(s, L)] = jnp.exp(-f0 / SQW) * smc
            parents_v[pl.ds(s, L)] = jnp.full((L,), goal_i, jnp.int32)
            path_v[pl.ds(s, L)] = (gmc > 0.5).astype(jnp.int32)
            return 0

        lax.fori_loop(0, HW // L, init_chunk, 0)

        nk = lane + (lane >= 4).astype(jnp.int32)
        dy = nk // 3 - 1
        dx = nk % 3 - 1
        nbr_lane = lane < 8

        def cond(carry):
            t, solved, _ = carry
            return jnp.logical_and(t < TMAX, jnp.logical_not(solved))

        def body(carry):
            t, _, t1 = carry

            def amax_chunk(c, st):
                bv, bi = st
                s = c * L
                v = val_v[pl.ds(s, L)]
                m = v > bv
                return jnp.where(m, v, bv), jnp.where(m, s + lane, bi)

            bv, bi = lax.fori_loop(
                0, HW // L, amax_chunk,
                (jnp.full((L,), -1.0, jnp.float32), jnp.zeros((L,), jnp.int32)))
            maxv = jnp.max(bv)
            ind = (-jnp.max(jnp.where(bv == maxv, -bi.astype(jnp.float32),
                                      -float(HW)))).astype(jnp.int32)
            indv = jnp.full((L,), ind, jnp.int32)
            solved = ind == goal_i

            plsc.store_scatter(hist_v, [indv], one_f, mask=lane0)
            rm = jnp.logical_and(lane0, jnp.logical_not(solved))
            plsc.store_scatter(open_v, [indv], zero_f, mask=rm)
            plsc.store_scatter(val_v, [indv], zero_f, mask=rm)

            gsel = plsc.load_gather(g_v, [indv])
            csel = plsc.load_gather(cm_v, [indv])
            new_g = jnp.max(gsel) + jnp.max(csel)
            new_gv = jnp.full((L,), new_g, jnp.float32)

            ny = (indv >> 5) + dy
            nx = (indv & 31) + dx
            valid = (nbr_lane & (ny >= 0) & (ny < H) & (nx >= 0) & (nx < W))
            nidx = jnp.where(valid, ny * W + nx, 0)
            gn = plsc.load_gather(g_v, [nidx], mask=valid)
            on = plsc.load_gather(open_v, [nidx], mask=valid)
            hn = plsc.load_gather(hist_v, [nidx], mask=valid)
            hh = plsc.load_gather(h_v, [nidx], mask=valid)
            upd = valid & (((on <= 0.5) & (hn <= 0.5))
                           | ((on > 0.5) & (gn > new_gv)))
            plsc.store_scatter(g_v, [nidx], new_gv, mask=upd)
            plsc.store_scatter(open_v, [nidx], one_f, mask=upd)
            plsc.store_scatter(parents_v, [nidx], indv, mask=upd)
            fn = 0.5 * new_gv + 0.5 * hh
            plsc.store_scatter(val_v, [nidx], jnp.exp(-fn / SQW), mask=upd)

            t1 = jnp.where(solved, t, t1)
            return t + 1, solved, t1

        _, _, t1 = lax.while_loop(
            cond, body,
            (jnp.int32(0), jnp.bool_(False), jnp.int32(TMAX - 1)))

        goalv = jnp.full((L,), goal_i, jnp.int32)
        loc0 = jnp.max(plsc.load_gather(parents_v,
                                        [goalv]).astype(jnp.float32))

        def bt(_, loc):
            locv = jnp.full((L,), loc.astype(jnp.int32), jnp.int32)
            plsc.store_scatter(path_v, [locv], one_i, mask=lane0)
            return jnp.max(plsc.load_gather(parents_v,
                                            [locv]).astype(jnp.float32))

        lax.fori_loop(0, t1, bt, loc0)

        pltpu.sync_copy(hist_v, hist_hbm.at[b])
        pltpu.sync_copy(path_v, path_hbm.at[b])


@jax.jit
def _astar_sc(cm, h, gm, sm):
    mesh = plsc.VectorSubcoreMesh(core_axis_name="c", subcore_axis_name="s",
                                  num_cores=NC, num_subcores=NS)
    f32 = jnp.float32
    run = pl.kernel(
        _astar_body,
        out_type=(jax.ShapeDtypeStruct((B, HW), f32),
                  jax.ShapeDtypeStruct((B, HW), jnp.int32)),
        mesh=mesh,
        compiler_params=pltpu.CompilerParams(needs_layout_passes=False),
        scratch_types=(
            pltpu.VMEM((HW,), f32),
            pltpu.VMEM((HW,), f32),
            pltpu.VMEM((HW,), f32),
            pltpu.VMEM((HW,), f32),
            pltpu.VMEM((HW,), f32),
            pltpu.VMEM((HW,), f32),
            pltpu.VMEM((HW,), f32),
            pltpu.VMEM((HW,), f32),
            pltpu.VMEM((HW,), jnp.int32),
            pltpu.VMEM((HW,), jnp.int32),
        ),
    )
    return run(cm, h, gm, sm)


def _heuristic(gm):
    gy, gx = jnp.meshgrid(jnp.arange(H), jnp.arange(W), indexing="ij")
    loc = jnp.stack([gy, gx], 0).astype(gm.dtype)
    loc_e = loc.reshape(2, -1)[None]
    goal_loc = jnp.einsum("kij,bij->bk", loc, gm)[:, :, None]
    dxdy = jnp.abs(loc_e - goal_loc)
    h = dxdy.sum(1) - dxdy.min(1)
    euc = jnp.sqrt(((loc_e - goal_loc) ** 2).sum(1))
    return h + 0.001 * euc


def kernel(cost_maps, start_maps, goal_maps, heuristic_maps, obstacles_maps):
    cm = cost_maps[:, 0].reshape(B, HW)
    gm2 = goal_maps[:, 0]
    gm = gm2.reshape(B, HW)
    sm = start_maps[:, 0].reshape(B, HW)
    h = _heuristic(gm2)
    hist, path = _astar_sc(cm, h, gm, sm)
    return hist.reshape(B, 1, H, W), path.reshape(B, 1, H, W)


# speedup vs baseline: 920.8259x; 1.1464x over previous
import math

import jax
import jax.numpy as jnp
from jax import lax
from jax.experimental import pallas as pl
from jax.experimental.pallas import tpu as pltpu
from jax.experimental.pallas import tpu_sc as plsc

B, H, W = 8, 32, 32
HW = H * W
TMAX = HW
SQW = math.sqrt(W)
NC, NS = 1, 16
L = 16
CPR = W // L


def _astar_body(cm_hbm, h_hbm, gm_hbm, sm_hbm, hist_hbm, path_hbm,
                cm_v, h_v, gm_v, sm_v, val_v, g_v, open_v, hist_v,
                parents_v, path_v):
    wid = lax.axis_index("s") * NC + lax.axis_index("c")

    @pl.when(wid < B)
    def _():
        b = wid
        pltpu.sync_copy(cm_hbm.at[b], cm_v)
        pltpu.sync_copy(h_hbm.at[b], h_v)
        pltpu.sync_copy(gm_hbm.at[b], gm_v)
        pltpu.sync_copy(sm_hbm.at[b], sm_v)

        lane = lax.iota(jnp.int32, L)
        zero_f = jnp.zeros((L,), jnp.float32)
        one_f = jnp.ones((L,), jnp.float32)
        one_i = jnp.ones((L,), jnp.int32)
        lane0 = lane == 0

        def find_chunk(c, acc):
            s = c * L
            gmc = gm_v[pl.ds(s, L)]
            smc = sm_v[pl.ds(s, L)]
            negidx = -(s + lane).astype(jnp.float32)
            gacc, sacc = acc
            gcand = jnp.max(jnp.where(gmc > 0.5, negidx, -float(HW)))
            scand = jnp.max(jnp.where(smc > 0.5, negidx, -float(HW)))
            return jnp.maximum(gacc, gcand), jnp.maximum(sacc, scand)

        gneg, sneg = lax.fori_loop(0, HW // L, find_chunk,
                                   (jnp.float32(-HW), jnp.float32(-HW)),
                                   unroll=4)
        goal_i = (-gneg).astype(jnp.int32)
        start_i = (-sneg).astype(jnp.int32)

        goal_fill = jnp.full((L,), goal_i, jnp.int32)
        zero_i = jnp.zeros((L,), jnp.int32)

        def init_chunk(c, _):
            s = c * L
            smc = sm_v[pl.ds(s, L)]
            g_v[pl.ds(s, L)] = zero_f
            hist_v[pl.ds(s, L)] = zero_f
            open_v[pl.ds(s, L)] = smc
            val_v[pl.ds(s, L)] = zero_f
            parents_v[pl.ds(s, L)] = goal_fill
            path_v[pl.ds(s, L)] = zero_i
            return 0

        lax.fori_loop(0, HW // L, init_chunk, 0, unroll=4)

        startv = jnp.full((L,), start_i, jnp.int32)
        hs = plsc.load_gather(h_v, [startv])
        plsc.store_scatter(val_v, [startv], jnp.exp(-(0.5 * hs) / SQW),
                           mask=lane0)
        goalv = jnp.full((L,), goal_i, jnp.int32)
        plsc.store_scatter(path_v, [goalv], one_i, mask=lane0)

        nk = lane + (lane >= 4).astype(jnp.int32)
        dy = nk // 3 - 1
        dx = nk % 3 - 1
        nbr_lane = lane < 8

        def cond(carry):
            t, solved = carry[0], carry[1]
            return jnp.logical_and(t < TMAX, jnp.logical_not(solved))

        def body(carry):
            t, _, t1, ymin, ymax = carry

            def acond(st):
                return st[0] < (ymax + 1) * CPR

            def achunk(st):
                c, bv, bi = st
                s = c * L
                v = val_v[pl.ds(s, L)]
                m = v > bv
                return (c + 1, jnp.where(m, v, bv),
                        jnp.where(m, s + lane, bi))

            _, bv, bi = lax.while_loop(
                acond, achunk,
                (ymin * CPR, jnp.full((L,), -1.0, jnp.float32),
                 jnp.zeros((L,), jnp.int32)))
            maxv = jnp.max(bv)
            ind = (-jnp.max(jnp.where(bv == maxv, -bi.astype(jnp.float32),
                                      -float(HW)))).astype(jnp.int32)
            indv = jnp.full((L,), ind, jnp.int32)
            solved = ind == goal_i

            plsc.store_scatter(hist_v, [indv], one_f, mask=lane0)
            rm = jnp.logical_and(lane0, jnp.logical_not(solved))
            plsc.store_scatter(open_v, [indv], zero_f, mask=rm)
            plsc.store_scatter(val_v, [indv], zero_f, mask=rm)

            new_gv = plsc.load_gather(g_v, [indv]) + plsc.load_gather(
                cm_v, [indv])

            iy = ind >> 5
            ny = (indv >> 5) + dy
            nx = (indv & 31) + dx
            valid = (nbr_lane & (ny >= 0) & (ny < H) & (nx >= 0) & (nx < W))
            nidx = jnp.where(valid, ny * W + nx, 0)
            gn = plsc.load_gather(g_v, [nidx], mask=valid)
            on = plsc.load_gather(open_v, [nidx], mask=valid)
            hn = plsc.load_gather(hist_v, [nidx], mask=valid)
            hh = plsc.load_gather(h_v, [nidx], mask=valid)
            upd = valid & (((on <= 0.5) & (hn <= 0.5))
                           | ((on > 0.5) & (gn > new_gv)))
            plsc.store_scatter(g_v, [nidx], new_gv, mask=upd)
            plsc.store_scatter(open_v, [nidx], one_f, mask=upd)
            plsc.store_scatter(parents_v, [nidx], indv, mask=upd)
            fn = 0.5 * new_gv + 0.5 * hh
            plsc.store_scatter(val_v, [nidx], jnp.exp(-fn / SQW), mask=upd)

            t1 = jnp.where(solved, t, t1)
            ymin = jnp.minimum(ymin, jnp.maximum(iy - 1, 0))
            ymax = jnp.maximum(ymax, jnp.minimum(iy + 1, H - 1))
            return t + 1, solved, t1, ymin, ymax

        sy = start_i >> 5
        _, _, t1, _, _ = lax.while_loop(
            cond, body,
            (jnp.int32(0), jnp.bool_(False), jnp.int32(TMAX - 1), sy, sy))

        loc0 = jnp.max(plsc.load_gather(parents_v,
                                        [goalv]).astype(jnp.float32))

        def bt(_, loc):
            locv = jnp.full((L,), loc.astype(jnp.int32), jnp.int32)
            plsc.store_scatter(path_v, [locv], one_i, mask=lane0)
            return jnp.max(plsc.load_gather(parents_v,
                                            [locv]).astype(jnp.float32))

        lax.fori_loop(0, t1, bt, loc0)

        pltpu.sync_copy(hist_v, hist_hbm.at[b])
        pltpu.sync_copy(path_v, path_hbm.at[b])


@jax.jit
def _astar_sc(cm, h, gm, sm):
    mesh = plsc.VectorSubcoreMesh(core_axis_name="c", subcore_axis_name="s",
                                  num_cores=NC, num_subcores=NS)
    f32 = jnp.float32
    run = pl.kernel(
        _astar_body,
        out_type=(jax.ShapeDtypeStruct((B, HW), f32),
                  jax.ShapeDtypeStruct((B, HW), jnp.int32)),
        mesh=mesh,
        compiler_params=pltpu.CompilerParams(needs_layout_passes=False),
        scratch_types=(
            pltpu.VMEM((HW,), f32),
            pltpu.VMEM((HW,), f32),
            pltpu.VMEM((HW,), f32),
            pltpu.VMEM((HW,), f32),
            pltpu.VMEM((HW,), f32),
            pltpu.VMEM((HW,), f32),
            pltpu.VMEM((HW,), f32),
            pltpu.VMEM((HW,), f32),
            pltpu.VMEM((HW,), jnp.int32),
            pltpu.VMEM((HW,), jnp.int32),
        ),
    )
    return run(cm, h, gm, sm)


def _heuristic(gm):
    gy, gx = jnp.meshgrid(jnp.arange(H), jnp.arange(W), indexing="ij")
    loc = jnp.stack([gy, gx], 0).astype(gm.dtype)
    loc_e = loc.reshape(2, -1)[None]
    goal_loc = jnp.einsum("kij,bij->bk", loc, gm)[:, :, None]
    dxdy = jnp.abs(loc_e - goal_loc)
    h = dxdy.sum(1) - dxdy.min(1)
    euc = jnp.sqrt(((loc_e - goal_loc) ** 2).sum(1))
    return h + 0.001 * euc


def kernel(cost_maps, start_maps, goal_maps, heuristic_maps, obstacles_maps):
    cm = cost_maps[:, 0].reshape(B, HW)
    gm2 = goal_maps[:, 0]
    gm = gm2.reshape(B, HW)
    sm = start_maps[:, 0].reshape(B, HW)
    h = _heuristic(gm2)
    hist, path = _astar_sc(cm, h, gm, sm)
    return hist.reshape(B, 1, H, W), path.reshape(B, 1, H, W)


# speedup vs baseline: 922.0036x; 1.0013x over previous
import math

import jax
import jax.numpy as jnp
from jax import lax
from jax.experimental import pallas as pl
from jax.experimental.pallas import tpu as pltpu
from jax.experimental.pallas import tpu_sc as plsc

B, H, W = 8, 32, 32
HW = H * W
TMAX = HW
SQW = math.sqrt(W)
NC, NS = 1, 16
L = 16
CPR = W // L


def _astar_body(cm_hbm, h_hbm, gm_hbm, sm_hbm, hist_hbm, path_hbm,
                cm_v, h_v, gm_v, sm_v, val_v, g_v, open_v, hist_v,
                parents_v, path_v):
    wid = lax.axis_index("s") * NC + lax.axis_index("c")

    @pl.when(wid < B)
    def _():
        b = wid
        pltpu.sync_copy(cm_hbm.at[b], cm_v)
        pltpu.sync_copy(h_hbm.at[b], h_v)
        pltpu.sync_copy(gm_hbm.at[b], gm_v)
        pltpu.sync_copy(sm_hbm.at[b], sm_v)

        lane = lax.iota(jnp.int32, L)
        zero_f = jnp.zeros((L,), jnp.float32)
        one_f = jnp.ones((L,), jnp.float32)
        one_i = jnp.ones((L,), jnp.int32)
        lane0 = lane == 0

        def find_chunk(c, acc):
            s = c * L
            gmc = gm_v[pl.ds(s, L)]
            smc = sm_v[pl.ds(s, L)]
            negidx = -(s + lane).astype(jnp.float32)
            gacc, sacc = acc
            gcand = jnp.max(jnp.where(gmc > 0.5, negidx, -float(HW)))
            scand = jnp.max(jnp.where(smc > 0.5, negidx, -float(HW)))
            return jnp.maximum(gacc, gcand), jnp.maximum(sacc, scand)

        gneg, sneg = lax.fori_loop(0, HW // L, find_chunk,
                                   (jnp.float32(-HW), jnp.float32(-HW)))
        goal_i = (-gneg).astype(jnp.int32)
        start_i = (-sneg).astype(jnp.int32)

        goal_fill = jnp.full((L,), goal_i, jnp.int32)
        zero_i = jnp.zeros((L,), jnp.int32)

        def init_chunk(c, _):
            s = c * L
            smc = sm_v[pl.ds(s, L)]
            g_v[pl.ds(s, L)] = zero_f
            hist_v[pl.ds(s, L)] = zero_f
            open_v[pl.ds(s, L)] = smc
            val_v[pl.ds(s, L)] = zero_f
            parents_v[pl.ds(s, L)] = goal_fill
            path_v[pl.ds(s, L)] = zero_i
            return 0

        lax.fori_loop(0, HW // L, init_chunk, 0)

        startv = jnp.full((L,), start_i, jnp.int32)
        hs = plsc.load_gather(h_v, [startv])
        plsc.store_scatter(val_v, [startv], jnp.exp(-(0.5 * hs) / SQW),
                           mask=lane0)
        goalv = jnp.full((L,), goal_i, jnp.int32)
        plsc.store_scatter(path_v, [goalv], one_i, mask=lane0)

        nk = lane + (lane >= 4).astype(jnp.int32)
        dy = nk // 3 - 1
        dx = nk % 3 - 1
        nbr_lane = lane < 8

        def cond(carry):
            t, solved = carry[0], carry[1]
            return jnp.logical_and(t < TMAX, jnp.logical_not(solved))

        def body(carry):
            t, _, t1, ymin, ymax = carry

            def acond(st):
                return st[0] < (ymax + 1) * CPR

            def achunk(st):
                c, bv, bi = st
                s = c * L
                v = val_v[pl.ds(s, L)]
                m = v > bv
                return (c + 1, jnp.where(m, v, bv),
                        jnp.where(m, s + lane, bi))

            _, bv, bi = lax.while_loop(
                acond, achunk,
                (ymin * CPR, jnp.full((L,), -1.0, jnp.float32),
                 jnp.zeros((L,), jnp.int32)))
            maxv = jnp.max(bv)
            ind = (-jnp.max(jnp.where(bv == maxv, -bi.astype(jnp.float32),
                                      -float(HW)))).astype(jnp.int32)
            indv = jnp.full((L,), ind, jnp.int32)
            solved = ind == goal_i

            plsc.store_scatter(hist_v, [indv], one_f, mask=lane0)
            rm = jnp.logical_and(lane0, jnp.logical_not(solved))
            plsc.store_scatter(open_v, [indv], zero_f, mask=rm)
            plsc.store_scatter(val_v, [indv], zero_f, mask=rm)

            new_gv = plsc.load_gather(g_v, [indv]) + plsc.load_gather(
                cm_v, [indv])

            iy = ind >> 5
            ny = (indv >> 5) + dy
            nx = (indv & 31) + dx
            valid = (nbr_lane & (ny >= 0) & (ny < H) & (nx >= 0) & (nx < W))
            nidx = jnp.where(valid, ny * W + nx, 0)
            gn = plsc.load_gather(g_v, [nidx], mask=valid)
            on = plsc.load_gather(open_v, [nidx], mask=valid)
            hn = plsc.load_gather(hist_v, [nidx], mask=valid)
            hh = plsc.load_gather(h_v, [nidx], mask=valid)
            upd = valid & (((on <= 0.5) & (hn <= 0.5))
                           | ((on > 0.5) & (gn > new_gv)))
            plsc.store_scatter(g_v, [nidx], new_gv, mask=upd)
            plsc.store_scatter(open_v, [nidx], one_f, mask=upd)
            plsc.store_scatter(parents_v, [nidx], indv, mask=upd)
            fn = 0.5 * new_gv + 0.5 * hh
            plsc.store_scatter(val_v, [nidx], jnp.exp(-fn / SQW), mask=upd)

            t1 = jnp.where(solved, t, t1)
            ymin = jnp.minimum(ymin, jnp.maximum(iy - 1, 0))
            ymax = jnp.maximum(ymax, jnp.minimum(iy + 1, H - 1))
            return t + 1, solved, t1, ymin, ymax

        sy = start_i >> 5
        _, _, t1, _, _ = lax.while_loop(
            cond, body,
            (jnp.int32(0), jnp.bool_(False), jnp.int32(TMAX - 1), sy, sy))

        loc0 = jnp.max(plsc.load_gather(parents_v,
                                        [goalv]).astype(jnp.float32))

        def bt(_, loc):
            locv = jnp.full((L,), loc.astype(jnp.int32), jnp.int32)
            plsc.store_scatter(path_v, [locv], one_i, mask=lane0)
            return jnp.max(plsc.load_gather(parents_v,
                                            [locv]).astype(jnp.float32))

        lax.fori_loop(0, t1, bt, loc0)

        pltpu.sync_copy(hist_v, hist_hbm.at[b])
        pltpu.sync_copy(path_v, path_hbm.at[b])


@jax.jit
def _astar_sc(cm, h, gm, sm):
    mesh = plsc.VectorSubcoreMesh(core_axis_name="c", subcore_axis_name="s",
                                  num_cores=NC, num_subcores=NS)
    f32 = jnp.float32
    run = pl.kernel(
        _astar_body,
        out_type=(jax.ShapeDtypeStruct((B, HW), f32),
                  jax.ShapeDtypeStruct((B, HW), jnp.int32)),
        mesh=mesh,
        compiler_params=pltpu.CompilerParams(needs_layout_passes=False),
        scratch_types=(
            pltpu.VMEM((HW,), f32),
            pltpu.VMEM((HW,), f32),
            pltpu.VMEM((HW,), f32),
            pltpu.VMEM((HW,), f32),
            pltpu.VMEM((HW,), f32),
            pltpu.VMEM((HW,), f32),
            pltpu.VMEM((HW,), f32),
            pltpu.VMEM((HW,), f32),
            pltpu.VMEM((HW,), jnp.int32),
            pltpu.VMEM((HW,), jnp.int32),
        ),
    )
    return run(cm, h, gm, sm)


def _heuristic(gm):
    gy, gx = jnp.meshgrid(jnp.arange(H), jnp.arange(W), indexing="ij")
    loc = jnp.stack([gy, gx], 0).astype(gm.dtype)
    loc_e = loc.reshape(2, -1)[None]
    goal_loc = jnp.einsum("kij,bij->bk", loc, gm)[:, :, None]
    dxdy = jnp.abs(loc_e - goal_loc)
    h = dxdy.sum(1) - dxdy.min(1)
    euc = jnp.sqrt(((loc_e - goal_loc) ** 2).sum(1))
    return h + 0.001 * euc


def kernel(cost_maps, start_maps, goal_maps, heuristic_maps, obstacles_maps):
    cm = cost_maps[:, 0].reshape(B, HW)
    gm2 = goal_maps[:, 0]
    gm = gm2.reshape(B, HW)
    sm = start_maps[:, 0].reshape(B, HW)
    h = _heuristic(gm2)
    hist, path = _astar_sc(cm, h, gm, sm)
    return hist.reshape(B, 1, H, W), path.reshape(B, 1, H, W)


# speedup vs baseline: 1060.0623x; 1.1497x over previous
import math

import jax
import jax.numpy as jnp
from jax import lax
from jax.experimental import pallas as pl
from jax.experimental.pallas import tpu as pltpu
from jax.experimental.pallas import tpu_sc as plsc

B, H, W = 8, 32, 32
HW = H * W
TMAX = HW
SQW = math.sqrt(W)
NC, NS = 1, 16
L = 16
CPR = W // L


def _astar_body(cm_hbm, h_hbm, gs_hbm, hist_hbm, path_hbm,
                cm_v, h_v, gs_v, val_v, g_v, open_v, hist_v,
                parents_v, path_v, sem1, sem2, sem3):
    wid = lax.axis_index("s") * NC + lax.axis_index("c")

    @pl.when(wid < B)
    def _():
        b = wid
        c1 = pltpu.async_copy(cm_hbm.at[b, 0], cm_v, sem1)
        c2 = pltpu.async_copy(h_hbm.at[b], h_v, sem2)
        c3 = pltpu.async_copy(gs_hbm, gs_v, sem3)

        lane = lax.iota(jnp.int32, L)
        zero_f = jnp.zeros((L,), jnp.float32)
        one_f = jnp.ones((L,), jnp.float32)
        one_i = jnp.ones((L,), jnp.int32)
        lane0 = lane == 0

        c3.wait()
        sel_b = lane == b
        goal_i = jnp.max(jnp.where(sel_b, gs_v[0, :], 0)
                         .astype(jnp.float32)).astype(jnp.int32)
        start_i = jnp.max(jnp.where(sel_b, gs_v[1, :], 0)
                          .astype(jnp.float32)).astype(jnp.int32)

        goal_fill = jnp.full((L,), goal_i, jnp.int32)
        zero_i = jnp.zeros((L,), jnp.int32)

        def init_chunk(c, _):
            s = c * L
            g_v[pl.ds(s, L)] = zero_f
            open_v[pl.ds(s, L)] = zero_f
            val_v[pl.ds(s, L)] = zero_f
            parents_v[pl.ds(s, L)] = goal_fill
            hist_v[c >> 1, pl.ds((c & 1) * L, L)] = zero_f
            path_v[c >> 1, pl.ds((c & 1) * L, L)] = zero_i
            return 0

        lax.fori_loop(0, HW // L, init_chunk, 0)

        startv = jnp.full((L,), start_i, jnp.int32)
        goalv = jnp.full((L,), goal_i, jnp.int32)
        c2.wait()
        hs = plsc.load_gather(h_v, [startv >> 5, startv & 31])
        plsc.store_scatter(open_v, [startv], one_f, mask=lane0)
        plsc.store_scatter(val_v, [startv], jnp.exp(-(0.5 * hs) / SQW),
                           mask=lane0)
        plsc.store_scatter(path_v, [goalv >> 5, goalv & 31], one_i,
                           mask=lane0)
        c1.wait()

        nk = lane + (lane >= 4).astype(jnp.int32)
        dy = nk // 3 - 1
        dx = nk % 3 - 1
        nbr_lane = lane < 8

        def cond(carry):
            t, solved = carry[0], carry[1]
            return jnp.logical_and(t < TMAX, jnp.logical_not(solved))

        def body(carry):
            t, _, t1, ymin, ymax = carry

            def acond(st):
                return st[0] < (ymax + 1) * CPR

            def achunk(st):
                c, bv, bi = st
                s = c * L
                v = val_v[pl.ds(s, L)]
                m = v > bv
                return (c + 1, jnp.where(m, v, bv),
                        jnp.where(m, s + lane, bi))

            _, bv, bi = lax.while_loop(
                acond, achunk,
                (ymin * CPR, jnp.full((L,), -1.0, jnp.float32),
                 jnp.zeros((L,), jnp.int32)))
            maxv = jnp.max(bv)
            ind = (-jnp.max(jnp.where(bv == maxv, -bi.astype(jnp.float32),
                                      -float(HW)))).astype(jnp.int32)
            indv = jnp.full((L,), ind, jnp.int32)
            solved = ind == goal_i

            iyv = indv >> 5
            ixv = indv & 31
            plsc.store_scatter(hist_v, [iyv, ixv], one_f, mask=lane0)
            rm = jnp.logical_and(lane0, jnp.logical_not(solved))
            plsc.store_scatter(open_v, [indv], zero_f, mask=rm)
            plsc.store_scatter(val_v, [indv], zero_f, mask=rm)

            new_gv = plsc.load_gather(g_v, [indv]) + plsc.load_gather(
                cm_v, [iyv, ixv])

            iy = ind >> 5
            ny = iyv + dy
            nx = ixv + dx
            valid = (nbr_lane & (ny >= 0) & (ny < H) & (nx >= 0) & (nx < W))
            nyc = jnp.where(valid, ny, 0)
            nxc = jnp.where(valid, nx, 0)
            nidx = nyc * W + nxc
            gn = plsc.load_gather(g_v, [nidx], mask=valid)
            on = plsc.load_gather(open_v, [nidx], mask=valid)
            hn = plsc.load_gather(hist_v, [nyc, nxc], mask=valid)
            hh = plsc.load_gather(h_v, [nyc, nxc], mask=valid)
            upd = valid & (((on <= 0.5) & (hn <= 0.5))
                           | ((on > 0.5) & (gn > new_gv)))
            plsc.store_scatter(g_v, [nidx], new_gv, mask=upd)
            plsc.store_scatter(open_v, [nidx], one_f, mask=upd)
            plsc.store_scatter(parents_v, [nidx], indv, mask=upd)
            fn = 0.5 * new_gv + 0.5 * hh
            plsc.store_scatter(val_v, [nidx], jnp.exp(-fn / SQW), mask=upd)

            t1 = jnp.where(solved, t, t1)
            ymin = jnp.minimum(ymin, jnp.maximum(iy - 1, 0))
            ymax = jnp.maximum(ymax, jnp.minimum(iy + 1, H - 1))
            return t + 1, solved, t1, ymin, ymax

        sy = start_i >> 5
        _, _, t1, _, _ = lax.while_loop(
            cond, body,
            (jnp.int32(0), jnp.bool_(False), jnp.int32(TMAX - 1), sy, sy))

        loc0 = jnp.max(plsc.load_gather(parents_v,
                                        [goalv]).astype(jnp.float32))

        def bt(_, loc):
            locv = jnp.full((L,), loc.astype(jnp.int32), jnp.int32)
            plsc.store_scatter(path_v, [locv >> 5, locv & 31], one_i,
                               mask=lane0)
            return jnp.max(plsc.load_gather(parents_v,
                                            [locv]).astype(jnp.float32))

        lax.fori_loop(0, t1, bt, loc0)

        o1 = pltpu.async_copy(hist_v, hist_hbm.at[b, 0], sem1)
        o2 = pltpu.async_copy(path_v, path_hbm.at[b, 0], sem2)
        o1.wait()
        o2.wait()


@jax.jit
def _astar_sc(cm4, h3, gs):
    mesh = plsc.VectorSubcoreMesh(core_axis_name="c", subcore_axis_name="s",
                                  num_cores=NC, num_subcores=NS)
    f32 = jnp.float32
    run = pl.kernel(
        _astar_body,
        out_type=(jax.ShapeDtypeStruct((B, 1, H, W), f32),
                  jax.ShapeDtypeStruct((B, 1, H, W), jnp.int32)),
        mesh=mesh,
        compiler_params=pltpu.CompilerParams(needs_layout_passes=False),
        scratch_types=(
            pltpu.VMEM((H, W), f32),
            pltpu.VMEM((H, W), f32),
            pltpu.VMEM((2, L), jnp.int32),
            pltpu.VMEM((HW,), f32),
            pltpu.VMEM((HW,), f32),
            pltpu.VMEM((HW,), f32),
            pltpu.VMEM((H, W), f32),
            pltpu.VMEM((HW,), jnp.int32),
            pltpu.VMEM((H, W), jnp.int32),
            pltpu.SemaphoreType.DMA,
            pltpu.SemaphoreType.DMA,
            pltpu.SemaphoreType.DMA,
        ),
    )
    return run(cm4, h3, gs)


def _heuristic(gm):
    gy, gx = jnp.meshgrid(jnp.arange(H), jnp.arange(W), indexing="ij")
    loc = jnp.stack([gy, gx], 0).astype(gm.dtype)
    loc_e = loc.reshape(2, -1)[None]
    goal_loc = jnp.einsum("kij,bij->bk", loc, gm)[:, :, None]
    dxdy = jnp.abs(loc_e - goal_loc)
    h = dxdy.sum(1) - dxdy.min(1)
    euc = jnp.sqrt(((loc_e - goal_loc) ** 2).sum(1))
    return (h + 0.001 * euc).reshape(B, H, W)


def kernel(cost_maps, start_maps, goal_maps, heuristic_maps, obstacles_maps):
    gm2 = goal_maps[:, 0]
    h = _heuristic(gm2)
    goal_idx = jnp.argmax(goal_maps.reshape(B, HW), -1).astype(jnp.int32)
    start_idx = jnp.argmax(start_maps.reshape(B, HW), -1).astype(jnp.int32)
    gs = jnp.stack([jnp.concatenate([goal_idx, goal_idx]),
                    jnp.concatenate([start_idx, start_idx])])
    return _astar_sc(cost_maps, h, gs)


# speedup vs baseline: 1083.6752x; 1.0223x over previous
import math

import jax
import jax.numpy as jnp
from jax import lax
from jax.experimental import pallas as pl
from jax.experimental.pallas import tpu as pltpu
from jax.experimental.pallas import tpu_sc as plsc

B, H, W = 8, 32, 32
HW = H * W
TMAX = HW
SQW = math.sqrt(W)
NC, NS = 1, 16
L = 16
CPR = W // L


def _astar_body(cm_hbm, h_hbm, gm_hbm, sm_hbm, hist_hbm, path_hbm,
                cm_v, h_v, gm_v, sm_v, val_v, g_v, open_v, hist_v,
                parents_v, path_v, sem1, sem2, sem3, sem4):
    wid = lax.axis_index("s") * NC + lax.axis_index("c")

    @pl.when(wid < B)
    def _():
        b = wid
        c1 = pltpu.async_copy(cm_hbm.at[b, 0], cm_v, sem1)
        c2 = pltpu.async_copy(h_hbm.at[b], h_v, sem2)
        c3 = pltpu.async_copy(gm_hbm.at[b, 0], gm_v, sem3)
        c4 = pltpu.async_copy(sm_hbm.at[b, 0], sm_v, sem4)

        lane = lax.iota(jnp.int32, L)
        zero_f = jnp.zeros((L,), jnp.float32)
        one_f = jnp.ones((L,), jnp.float32)
        one_i = jnp.ones((L,), jnp.int32)
        lane0 = lane == 0

        c3.wait()
        c4.wait()

        def find_chunk(c, acc):
            r = c >> 1
            half = (c & 1) * L
            gmc = gm_v[r, pl.ds(half, L)]
            smc = sm_v[r, pl.ds(half, L)]
            negidx = -(c * L + lane).astype(jnp.float32)
            gacc, sacc = acc
            gcand = jnp.max(jnp.where(gmc > 0.5, negidx, -float(HW)))
            scand = jnp.max(jnp.where(smc > 0.5, negidx, -float(HW)))
            return jnp.maximum(gacc, gcand), jnp.maximum(sacc, scand)

        gneg, sneg = lax.fori_loop(0, HW // L, find_chunk,
                                   (jnp.float32(-HW), jnp.float32(-HW)))
        goal_i = (-gneg).astype(jnp.int32)
        start_i = (-sneg).astype(jnp.int32)

        goal_fill = jnp.full((L,), goal_i, jnp.int32)
        zero_i = jnp.zeros((L,), jnp.int32)

        def init_chunk(c, _):
            s = c * L
            g_v[pl.ds(s, L)] = zero_f
            open_v[pl.ds(s, L)] = zero_f
            val_v[pl.ds(s, L)] = zero_f
            parents_v[pl.ds(s, L)] = goal_fill
            hist_v[c >> 1, pl.ds((c & 1) * L, L)] = zero_f
            path_v[c >> 1, pl.ds((c & 1) * L, L)] = zero_i
            return 0

        lax.fori_loop(0, HW // L, init_chunk, 0)

        startv = jnp.full((L,), start_i, jnp.int32)
        goalv = jnp.full((L,), goal_i, jnp.int32)
        c2.wait()
        hs = plsc.load_gather(h_v, [startv])
        plsc.store_scatter(open_v, [startv], one_f, mask=lane0)
        plsc.store_scatter(val_v, [startv], jnp.exp(-(0.5 * hs) / SQW),
                           mask=lane0)
        plsc.store_scatter(path_v, [goalv >> 5, goalv & 31], one_i,
                           mask=lane0)
        c1.wait()

        nk = lane + (lane >= 4).astype(jnp.int32)
        dy = nk // 3 - 1
        dx = nk % 3 - 1
        nbr_lane = lane < 8

        def cond(carry):
            t, solved = carry[0], carry[1]
            return jnp.logical_and(t < TMAX, jnp.logical_not(solved))

        def body(carry):
            t, _, t1, ymin, ymax = carry

            def acond(st):
                return st[0] < (ymax + 1) * CPR

            def achunk(st):
                c, bv, bi = st
                s = c * L
                v = val_v[pl.ds(s, L)]
                m = v > bv
                return (c + 1, jnp.where(m, v, bv),
                        jnp.where(m, s + lane, bi))

            _, bv, bi = lax.while_loop(
                acond, achunk,
                (ymin * CPR, jnp.full((L,), -1.0, jnp.float32),
                 jnp.zeros((L,), jnp.int32)))
            maxv = jnp.max(bv)
            ind = (-jnp.max(jnp.where(bv == maxv, -bi.astype(jnp.float32),
                                      -float(HW)))).astype(jnp.int32)
            indv = jnp.full((L,), ind, jnp.int32)
            solved = ind == goal_i

            iyv = indv >> 5
            ixv = indv & 31
            plsc.store_scatter(hist_v, [iyv, ixv], one_f, mask=lane0)
            rm = jnp.logical_and(lane0, jnp.logical_not(solved))
            plsc.store_scatter(open_v, [indv], zero_f, mask=rm)
            plsc.store_scatter(val_v, [indv], zero_f, mask=rm)

            new_gv = plsc.load_gather(g_v, [indv]) + plsc.load_gather(
                cm_v, [iyv, ixv])

            iy = ind >> 5
            ny = iyv + dy
            nx = ixv + dx
            valid = (nbr_lane & (ny >= 0) & (ny < H) & (nx >= 0) & (nx < W))
            nyc = jnp.where(valid, ny, 0)
            nxc = jnp.where(valid, nx, 0)
            nidx = nyc * W + nxc
            gn = plsc.load_gather(g_v, [nidx], mask=valid)
            on = plsc.load_gather(open_v, [nidx], mask=valid)
            hn = plsc.load_gather(hist_v, [nyc, nxc], mask=valid)
            hh = plsc.load_gather(h_v, [nidx], mask=valid)
            upd = valid & (((on <= 0.5) & (hn <= 0.5))
                           | ((on > 0.5) & (gn > new_gv)))
            plsc.store_scatter(g_v, [nidx], new_gv, mask=upd)
            plsc.store_scatter(open_v, [nidx], one_f, mask=upd)
            plsc.store_scatter(parents_v, [nidx], indv, mask=upd)
            fn = 0.5 * new_gv + 0.5 * hh
            plsc.store_scatter(val_v, [nidx], jnp.exp(-fn / SQW), mask=upd)

            t1 = jnp.where(solved, t, t1)
            ymin = jnp.minimum(ymin, jnp.maximum(iy - 1, 0))
            ymax = jnp.maximum(ymax, jnp.minimum(iy + 1, H - 1))
            return t + 1, solved, t1, ymin, ymax

        sy = start_i >> 5
        _, _, t1, _, _ = lax.while_loop(
            cond, body,
            (jnp.int32(0), jnp.bool_(False), jnp.int32(TMAX - 1), sy, sy))

        loc0 = jnp.max(plsc.load_gather(parents_v,
                                        [goalv]).astype(jnp.float32))

        def bt(_, loc):
            locv = jnp.full((L,), loc.astype(jnp.int32), jnp.int32)
            plsc.store_scatter(path_v, [locv >> 5, locv & 31], one_i,
                               mask=lane0)
            return jnp.max(plsc.load_gather(parents_v,
                                            [locv]).astype(jnp.float32))

        lax.fori_loop(0, t1, bt, loc0)

        o1 = pltpu.async_copy(hist_v, hist_hbm.at[b, 0], sem1)
        o2 = pltpu.async_copy(path_v, path_hbm.at[b, 0], sem2)
        o1.wait()
        o2.wait()


@jax.jit
def _astar_sc(cm4, h2, gm4, sm4):
    mesh = plsc.VectorSubcoreMesh(core_axis_name="c", subcore_axis_name="s",
                                  num_cores=NC, num_subcores=NS)
    f32 = jnp.float32
    run = pl.kernel(
        _astar_body,
        out_type=(jax.ShapeDtypeStruct((B, 1, H, W), f32),
                  jax.ShapeDtypeStruct((B, 1, H, W), jnp.int32)),
        mesh=mesh,
        compiler_params=pltpu.CompilerParams(needs_layout_passes=False),
        scratch_types=(
            pltpu.VMEM((H, W), f32),
            pltpu.VMEM((HW,), f32),
            pltpu.VMEM((H, W), f32),
            pltpu.VMEM((H, W), f32),
            pltpu.VMEM((HW,), f32),
            pltpu.VMEM((HW,), f32),
            pltpu.VMEM((HW,), f32),
            pltpu.VMEM((H, W), f32),
            pltpu.VMEM((HW,), jnp.int32),
            pltpu.VMEM((H, W), jnp.int32),
            pltpu.SemaphoreType.DMA,
            pltpu.SemaphoreType.DMA,
            pltpu.SemaphoreType.DMA,
            pltpu.SemaphoreType.DMA,
        ),
    )
    return run(cm4, h2, gm4, sm4)


def _heuristic(gm):
    gy, gx = jnp.meshgrid(jnp.arange(H), jnp.arange(W), indexing="ij")
    loc = jnp.stack([gy, gx], 0).astype(gm.dtype)
    loc_e = loc.reshape(2, -1)[None]
    goal_loc = jnp.einsum("kij,bij->bk", loc, gm)[:, :, None]
    dxdy = jnp.abs(loc_e - goal_loc)
    h = dxdy.sum(1) - dxdy.min(1)
    euc = jnp.sqrt(((loc_e - goal_loc) ** 2).sum(1))
    return h + 0.001 * euc


def kernel(cost_maps, start_maps, goal_maps, heuristic_maps, obstacles_maps):
    h = _heuristic(goal_maps[:, 0])
    return _astar_sc(cost_maps, h, goal_maps, start_maps)
